# Initial kernel scaffold; baseline (speedup 1.0000x reference)
import jax
import jax.numpy as jnp
from jax.experimental import pallas as pl


def kernel(x, edge_index, W1, b1, W2, b2, W3, b3):
    raise NotImplementedError("write your pallas kernel here")


# speedup vs baseline: 11.4724x; 11.4724x over previous
import functools

import jax
import jax.numpy as jnp
from jax import lax
from jax.experimental import pallas as pl
from jax.experimental.pallas import tpu as pltpu
from jax.experimental.pallas import tpu_sc as plsc

NC = 2
NS = 16
NW = NC * NS
L = 16
K = 128
DW = 16
BM = 2048


def _sc_mesh():
    return plsc.VectorSubcoreMesh(
        core_axis_name="c", subcore_axis_name="s", num_cores=NC, num_subcores=NS
    )


_SC_PARAMS = pltpu.CompilerParams(use_tc_tiling_on_sc=False)


def _make_deg_kernel(nchunks, npad):
    rows_per_tile = npad // NS

    @functools.partial(
        pl.kernel,
        mesh=_sc_mesh(),
        compiler_params=_SC_PARAMS,
        out_type=jax.ShapeDtypeStruct((NC, npad, DW), jnp.float32),
        scratch_types=[
            pltpu.VMEM((K,), jnp.int32),
            pltpu.VMEM((K, DW), jnp.float32),
            pltpu.VMEM((K, DW), jnp.float32),
            pltpu.VMEM_SHARED((npad, DW), jnp.float32),
        ],
    )
    def deg_kernel(dst_hbm, out_hbm, didx, ones_v, zeros_v, accum):
        c = lax.axis_index("c")
        s = lax.axis_index("s")
        wid = c * NS + s
        base = s * rows_per_tile

        def fill_row(i, _):
            def fill_col(t, __):
                ones_v[i, pl.ds(t * L, L)] = jnp.full((L,), 1.0, jnp.float32)
                zeros_v[i, pl.ds(t * L, L)] = jnp.zeros((L,), jnp.float32)
                return __
            return lax.fori_loop(0, DW // L, fill_col, _)

        lax.fori_loop(0, K, fill_row, 0)

        def zero_stripe(t, _):
            pltpu.sync_copy(zeros_v, accum.at[pl.ds(base + t * K, K)])
            return _

        lax.fori_loop(0, rows_per_tile // K, zero_stripe, 0)
        plsc.subcore_barrier()

        def chunk(j, _):
            pltpu.sync_copy(dst_hbm.at[wid, j], didx)
            pltpu.sync_copy(ones_v, accum.at[didx], add=True)
            return _

        lax.fori_loop(0, nchunks, chunk, 0)
        plsc.subcore_barrier()

        def writeback(t, _):
            pltpu.sync_copy(
                accum.at[pl.ds(base + t * K, K)],
                out_hbm.at[c, pl.ds(base + t * K, K)],
            )
            return _

        lax.fori_loop(0, rows_per_tile // K, writeback, 0)

    return deg_kernel


def _make_spmm_kernel(d, nchunks, npad):
    rows_per_tile = npad // NS

    @functools.partial(
        pl.kernel,
        mesh=_sc_mesh(),
        compiler_params=_SC_PARAMS,
        out_type=jax.ShapeDtypeStruct((NC, npad, d), jnp.float32),
        scratch_types=[
            pltpu.VMEM((K,), jnp.int32),
            pltpu.VMEM((K,), jnp.int32),
            pltpu.VMEM((K, d), jnp.float32),
            pltpu.VMEM((K, d), jnp.float32),
            pltpu.VMEM_SHARED((npad, d), jnp.float32),
            pltpu.SemaphoreType.DMA,
        ],
    )
    def spmm_kernel(hp_hbm, src_hbm, dst_hbm, out_hbm,
                    sidx, didx, rows, zeros_v, accum, sem):
        c = lax.axis_index("c")
        s = lax.axis_index("s")
        wid = c * NS + s
        base = s * rows_per_tile

        def zero_row(i, _):
            def zero_col(t, __):
                zeros_v[i, pl.ds(t * L, L)] = jnp.zeros((L,), jnp.float32)
                return __
            return lax.fori_loop(0, d // L, zero_col, _)

        lax.fori_loop(0, K, zero_row, 0)

        def zero_stripe(t, _):
            pltpu.sync_copy(zeros_v, accum.at[pl.ds(base + t * K, K)])
            return _

        lax.fori_loop(0, rows_per_tile // K, zero_stripe, 0)
        plsc.subcore_barrier()

        def chunk(j, _):
            pltpu.sync_copy(src_hbm.at[wid, j], sidx)
            pltpu.sync_copy(dst_hbm.at[wid, j], didx)
            pltpu.async_copy(hp_hbm.at[sidx], rows, sem).wait()
            pltpu.sync_copy(rows, accum.at[didx], add=True)
            return _

        lax.fori_loop(0, nchunks, chunk, 0)
        plsc.subcore_barrier()

        def writeback(t, _):
            pltpu.sync_copy(
                accum.at[pl.ds(base + t * K, K)],
                out_hbm.at[c, pl.ds(base + t * K, K)],
            )
            return _

        lax.fori_loop(0, rows_per_tile // K, writeback, 0)

    return spmm_kernel


def _dot(a, b):
    return jax.lax.dot_general(
        a, b, (((1,), (0,)), ((), ())),
        precision=jax.lax.Precision.HIGHEST,
        preferred_element_type=jnp.float32,
    )


def _tc_first(x_pad, W1, deg2, npad, d_in, d_out):
    grid = (npad // BM,)

    def body(x_ref, w_ref, deg_ref, h_ref, dv_ref):
        deg = deg_ref[0, :, 0] + deg_ref[1, :, 0] + 1.0
        dinv = lax.rsqrt(deg)[:, None]
        dv_ref[...] = jnp.broadcast_to(dinv, (BM, 128))
        h_ref[...] = _dot(x_ref[...], w_ref[...]) * dinv

    return pl.pallas_call(
        body,
        grid=grid,
        in_specs=[
            pl.BlockSpec((BM, d_in), lambda i: (i, 0)),
            pl.BlockSpec((d_in, d_out), lambda i: (0, 0)),
            pl.BlockSpec((NC, BM, DW), lambda i: (0, i, 0)),
        ],
        out_specs=[
            pl.BlockSpec((BM, d_out), lambda i: (i, 0)),
            pl.BlockSpec((BM, 128), lambda i: (i, 0)),
        ],
        out_shape=[
            jax.ShapeDtypeStruct((npad, d_out), jnp.float32),
            jax.ShapeDtypeStruct((npad, 128), jnp.float32),
        ],
    )(x_pad, W1, deg2)


def _tc_mid(S, hp, dinv_col, b, W, npad, d_in, d_out):
    grid = (npad // BM,)

    def body(s_ref, h_ref, dv_ref, b_ref, w_ref, o_ref):
        dv = dv_ref[:, :1]
        acc = s_ref[0] + s_ref[1] + h_ref[...]
        xv = dv * acc + b_ref[...]
        xv = jnp.where(xv >= 0, xv, 0.2 * xv)
        o_ref[...] = _dot(xv, w_ref[...]) * dv

    return pl.pallas_call(
        body,
        grid=grid,
        in_specs=[
            pl.BlockSpec((NC, BM, d_in), lambda i: (0, i, 0)),
            pl.BlockSpec((BM, d_in), lambda i: (i, 0)),
            pl.BlockSpec((BM, 128), lambda i: (i, 0)),
            pl.BlockSpec((1, d_in), lambda i: (0, 0)),
            pl.BlockSpec((d_in, d_out), lambda i: (0, 0)),
        ],
        out_specs=pl.BlockSpec((BM, d_out), lambda i: (i, 0)),
        out_shape=jax.ShapeDtypeStruct((npad, d_out), jnp.float32),
    )(S, hp, dinv_col, b, W)


def _tc_last(S, hp, dinv_col, b, npad, d):
    grid = (npad // BM,)

    def body(s_ref, h_ref, dv_ref, b_ref, o_ref):
        dv = dv_ref[:, :1]
        acc = s_ref[0] + s_ref[1] + h_ref[...]
        o_ref[...] = jnp.tanh(dv * acc + b_ref[...])

    return pl.pallas_call(
        body,
        grid=grid,
        in_specs=[
            pl.BlockSpec((NC, BM, d), lambda i: (0, i, 0)),
            pl.BlockSpec((BM, d), lambda i: (i, 0)),
            pl.BlockSpec((BM, 128), lambda i: (i, 0)),
            pl.BlockSpec((1, d), lambda i: (0, 0)),
        ],
        out_specs=pl.BlockSpec((BM, d), lambda i: (i, 0)),
        out_shape=jax.ShapeDtypeStruct((npad, d), jnp.float32),
    )(S, hp, dinv_col, b)


def kernel(x, edge_index, W1, b1, W2, b2, W3, b3):
    n, d_in = x.shape
    d_hid = W1.shape[1]
    e = edge_index.shape[1]

    npad = -(-n // (NS * K)) * (NS * K)
    nchunks = -(-e // (NW * K))
    e_pad = nchunks * NW * K

    ei = edge_index.astype(jnp.int32)
    pad = jnp.full((e_pad - e,), n, jnp.int32)
    src3 = jnp.concatenate([ei[0], pad]).reshape(NW, nchunks, K)
    dst3 = jnp.concatenate([ei[1], pad]).reshape(NW, nchunks, K)
    x_pad = jnp.pad(x, ((0, npad - n), (0, 0)))

    deg2 = _make_deg_kernel(nchunks, npad)(dst3)
    h1p, dinv_col = _tc_first(x_pad, W1, deg2, npad, d_in, d_hid)
    S1 = _make_spmm_kernel(d_hid, nchunks, npad)(h1p, src3, dst3)
    h2p = _tc_mid(S1, h1p, dinv_col, b1.reshape(1, -1), W2, npad, d_hid, d_hid)
    S2 = _make_spmm_kernel(d_hid, nchunks, npad)(h2p, src3, dst3)
    h3p = _tc_mid(S2, h2p, dinv_col, b2.reshape(1, -1), W3, npad, d_hid, d_in)
    S3 = _make_spmm_kernel(d_in, nchunks, npad)(h3p, src3, dst3)
    out = _tc_last(S3, h3p, dinv_col, b3.reshape(1, -1), npad, d_in)
    return out[:n]
